# Initial kernel scaffold; baseline (speedup 1.0000x reference)
import jax
import jax.numpy as jnp
from jax.experimental import pallas as pl


def kernel(x, edge_index, batch, W1, b1, g1, be1, W2, b2, g2, be2, Wl, bl, gl, bel, Wf, bf, gf, bef):
    raise NotImplementedError("write your pallas kernel here")


# speedup vs baseline: 2.9020x; 2.9020x over previous
import jax
import jax.numpy as jnp
from jax.experimental import pallas as pl

_N = 10000
_G = 64
_EPS = 1e-5
_ROWS = 1000


def _mm_stats_body(a_ref, w_ref, b_ref, y_ref, acc_ref):
    y = jnp.dot(a_ref[...], w_ref[...], preferred_element_type=jnp.float32)
    y = y + b_ref[...][None, :]
    y_ref[...] = y

    @pl.when(pl.program_id(0) == 0)
    def _():
        acc_ref[...] = jnp.zeros_like(acc_ref)

    acc_ref[0:1, :] += jnp.sum(y, axis=0, keepdims=True)
    acc_ref[1:2, :] += jnp.sum(y * y, axis=0, keepdims=True)


def _mm_stats(a, w, b):
    n, d = a.shape[0], w.shape[1]
    return pl.pallas_call(
        _mm_stats_body,
        grid=(n // _ROWS,),
        in_specs=[
            pl.BlockSpec((_ROWS, a.shape[1]), lambda i: (i, 0)),
            pl.BlockSpec((a.shape[1], d), lambda i: (0, 0)),
            pl.BlockSpec((d,), lambda i: (0,)),
        ],
        out_specs=[
            pl.BlockSpec((_ROWS, d), lambda i: (i, 0)),
            pl.BlockSpec((8, d), lambda i: (0, 0)),
        ],
        out_shape=[
            jax.ShapeDtypeStruct((n, d), jnp.float32),
            jax.ShapeDtypeStruct((8, d), jnp.float32),
        ],
    )(a, w, b)


def _bn_tanh_mm_body(y_ref, acc_ref, g_ref, be_ref, dis_ref, w_ref, p_ref):
    m = acc_ref[0:1, :] / _N
    v = acc_ref[1:2, :] / _N - m * m
    h = jnp.tanh((y_ref[...] - m) * jax.lax.rsqrt(v + _EPS) * g_ref[...][None, :]
                 + be_ref[...][None, :])
    hd = h * dis_ref[...]
    p_ref[...] = jnp.dot(hd, w_ref[...], preferred_element_type=jnp.float32)


def _bn_tanh_mm(y, acc, g, be, dis, w):
    n, d = y.shape
    do = w.shape[1]
    return pl.pallas_call(
        _bn_tanh_mm_body,
        grid=(n // _ROWS,),
        in_specs=[
            pl.BlockSpec((_ROWS, d), lambda i: (i, 0)),
            pl.BlockSpec((8, d), lambda i: (0, 0)),
            pl.BlockSpec((d,), lambda i: (0,)),
            pl.BlockSpec((d,), lambda i: (0,)),
            pl.BlockSpec((_ROWS, 1), lambda i: (i, 0)),
            pl.BlockSpec((d, do), lambda i: (0, 0)),
        ],
        out_specs=pl.BlockSpec((_ROWS, do), lambda i: (i, 0)),
        out_shape=jax.ShapeDtypeStruct((n, do), jnp.float32),
    )(y, acc, g, be, dis, w)


def _combine_stats_body(agg_ref, p_ref, dis_ref, b_ref, y_ref, acc_ref):
    y = (agg_ref[...] + p_ref[...]) * dis_ref[...] + b_ref[...][None, :]
    y_ref[...] = y

    @pl.when(pl.program_id(0) == 0)
    def _():
        acc_ref[...] = jnp.zeros_like(acc_ref)

    acc_ref[0:1, :] += jnp.sum(y, axis=0, keepdims=True)
    acc_ref[1:2, :] += jnp.sum(y * y, axis=0, keepdims=True)


def _combine_stats(agg, p, dis, b):
    n, d = p.shape
    return pl.pallas_call(
        _combine_stats_body,
        grid=(n // _ROWS,),
        in_specs=[
            pl.BlockSpec((_ROWS, d), lambda i: (i, 0)),
            pl.BlockSpec((_ROWS, d), lambda i: (i, 0)),
            pl.BlockSpec((_ROWS, 1), lambda i: (i, 0)),
            pl.BlockSpec((d,), lambda i: (0,)),
        ],
        out_specs=[
            pl.BlockSpec((_ROWS, d), lambda i: (i, 0)),
            pl.BlockSpec((8, d), lambda i: (0, 0)),
        ],
        out_shape=[
            jax.ShapeDtypeStruct((n, d), jnp.float32),
            jax.ShapeDtypeStruct((8, d), jnp.float32),
        ],
    )(agg, p, dis, b)


def _bn_tanh_pool_body(y_ref, acc_ref, g_ref, be_ref, batch_ref, s_ref, cnt_ref):
    m = acc_ref[0:1, :] / _N
    v = acc_ref[1:2, :] / _N - m * m
    h = jnp.tanh((y_ref[...] - m) * jax.lax.rsqrt(v + _EPS) * g_ref[...][None, :]
                 + be_ref[...][None, :])
    onehot = (batch_ref[...] ==
              jax.lax.broadcasted_iota(jnp.int32, (1, _G), 1)).astype(jnp.float32)

    @pl.when(pl.program_id(0) == 0)
    def _():
        s_ref[...] = jnp.zeros_like(s_ref)
        cnt_ref[...] = jnp.zeros_like(cnt_ref)

    dims = (((0,), (0,)), ((), ()))
    s_ref[...] += jax.lax.dot_general(onehot, h, dims,
                                      preferred_element_type=jnp.float32)
    cnt_ref[...] += jax.lax.dot_general(
        onehot, jnp.ones((onehot.shape[0], 128), jnp.float32), dims,
        preferred_element_type=jnp.float32)


def _bn_tanh_pool(y, acc, g, be, batch):
    n, d = y.shape
    return pl.pallas_call(
        _bn_tanh_pool_body,
        grid=(n // _ROWS,),
        in_specs=[
            pl.BlockSpec((_ROWS, d), lambda i: (i, 0)),
            pl.BlockSpec((8, d), lambda i: (0, 0)),
            pl.BlockSpec((d,), lambda i: (0,)),
            pl.BlockSpec((d,), lambda i: (0,)),
            pl.BlockSpec((_ROWS, 1), lambda i: (i, 0)),
        ],
        out_specs=[
            pl.BlockSpec((_G, d), lambda i: (0, 0)),
            pl.BlockSpec((_G, 128), lambda i: (0, 0)),
        ],
        out_shape=[
            jax.ShapeDtypeStruct((_G, d), jnp.float32),
            jax.ShapeDtypeStruct((_G, 128), jnp.float32),
        ],
    )(y, acc, g, be, batch)


def _head_body(s_ref, cnt_ref, wl_ref, bl_ref, gl_ref, bel_ref,
               wf_ref, bf_ref, gf_ref, bef_ref, o_ref):
    p = s_ref[...] / jnp.maximum(cnt_ref[:, 0:1], 1.0)
    y = jnp.dot(p, wl_ref[...], preferred_element_type=jnp.float32)
    y = y + bl_ref[...][None, :]
    m = jnp.mean(y, axis=0, keepdims=True)
    v = jnp.mean(y * y, axis=0, keepdims=True) - m * m
    z = jnp.tanh((y - m) * jax.lax.rsqrt(v + _EPS) * gl_ref[...][None, :]
                 + bel_ref[...][None, :])
    o = jnp.dot(z, wf_ref[...], preferred_element_type=jnp.float32)
    o = o + bf_ref[...][None, :]
    m2 = jnp.mean(o, axis=0, keepdims=True)
    v2 = jnp.mean(o * o, axis=0, keepdims=True) - m2 * m2
    o = (o - m2) * jax.lax.rsqrt(v2 + _EPS) * gf_ref[...][None, :] + bef_ref[...][None, :]
    o = o - jnp.max(o, axis=1, keepdims=True)
    o_ref[...] = o - jnp.log(jnp.sum(jnp.exp(o), axis=1, keepdims=True))


def _head(s, cnt, Wl, bl, gl, bel, Wf, bf, gf, bef):
    args = (s, cnt, Wl, bl, gl, bel, Wf, bf, gf, bef)
    return pl.pallas_call(
        _head_body,
        in_specs=[pl.BlockSpec(a.shape, (lambda *_, nd=a.ndim: (0,) * nd))
                  for a in args],
        out_specs=pl.BlockSpec((_G, Wf.shape[1]), lambda: (0, 0)),
        out_shape=jax.ShapeDtypeStruct((_G, Wf.shape[1]), jnp.float32),
    )(*args)


def kernel(x, edge_index, batch, W1, b1, g1, be1, W2, b2, g2, be2,
           Wl, bl, gl, bel, Wf, bf, gf, bef):
    src = edge_index[0]
    dst = edge_index[1]
    deg = jnp.ones((_N,), jnp.float32).at[dst].add(1.0)
    dis = jax.lax.rsqrt(deg)
    dis2 = dis[:, None]
    batch2 = batch[:, None]

    xd = x * dis2
    aggx = jnp.zeros_like(x).at[dst].add(xd[src, :])
    ax = (aggx + xd) * dis2
    y1, acc1 = _mm_stats(ax, W1, b1)
    p2 = _bn_tanh_mm(y1, acc1, g1, be1, dis2, W2)
    agg2 = jnp.zeros_like(p2).at[dst].add(p2[src, :])
    y2, acc2 = _combine_stats(agg2, p2, dis2, b2)
    s, cnt = _bn_tanh_pool(y2, acc2, g2, be2, batch2)
    return _head(s, cnt, Wl, bl, gl, bel, Wf, bf, gf, bef)


# speedup vs baseline: 6.3240x; 2.1792x over previous
import functools

import jax
import jax.numpy as jnp
from jax import lax
from jax.experimental import pallas as pl
from jax.experimental.pallas import tpu as pltpu
from jax.experimental.pallas import tpu_sc as plsc

_N = 10000
_E = 160000
_G = 64
_EPS = 1e-5
_ROWS = 1000

_NSUB = 16
_BLK = 128
_NBLK = 80
_EPAD = _NSUB * _NBLK * _BLK
_IGRP = 16
_ACCROWS = _N + 16


def _seg_sum(vals, srcp, dstp):
    C = vals.shape[0]
    cpc = C // 2
    mesh = plsc.VectorSubcoreMesh(core_axis_name="c", subcore_axis_name="s")

    @functools.partial(
        pl.kernel, mesh=mesh,
        out_type=jax.ShapeDtypeStruct((C, _N, 128), jnp.float32),
        scratch_types=[
            pltpu.VMEM((_IGRP, _BLK), jnp.int32),
            pltpu.VMEM((_IGRP, _BLK), jnp.int32),
            pltpu.VMEM((_BLK, 128), jnp.float32),
            pltpu.VMEM((_BLK, 128), jnp.float32),
            pltpu.VMEM_SHARED((_ACCROWS, 128), jnp.float32),
            pltpu.SemaphoreType.DMA,
            pltpu.SemaphoreType.DMA,
        ])
    def k(vals_h, src_h, dst_h, out_h, src_v, dst_v, g0, g1, acc,
          sem0, sem1):
        cid = lax.axis_index("c")
        sid = lax.axis_index("s")

        for t in range(cpc):
            chunk = cid * cpc + t
            vc = vals_h.at[chunk]

            @pl.loop(0, _BLK)
            def _(r):
                @pl.loop(0, 128, step=16)
                def _(l):
                    g0[r, pl.ds(l, 16)] = jnp.zeros((16,), jnp.float32)

            @pl.loop(0, 4)
            def _(z):
                pltpu.sync_copy(g0, acc.at[pl.ds(sid * 626 + z * 128, 128)])

            pltpu.sync_copy(g0.at[pl.ds(0, 114)],
                            acc.at[pl.ds(sid * 626 + 512, 114)])
            plsc.subcore_barrier()

            @pl.loop(0, _NBLK // _IGRP)
            def _(gi):
                pltpu.sync_copy(src_h.at[sid].at[pl.ds(gi * _IGRP, _IGRP)],
                                src_v)
                pltpu.sync_copy(dst_h.at[sid].at[pl.ds(gi * _IGRP, _IGRP)],
                                dst_v)

                @pl.loop(0, _IGRP, step=2)
                def _(j):
                    c0 = pltpu.async_copy(vc.at[src_v.at[j]], g0, sem0)
                    c1 = pltpu.async_copy(vc.at[src_v.at[j + 1]], g1, sem1)
                    c0.wait()
                    pltpu.sync_copy(g0, acc.at[dst_v.at[j]], add=True)
                    c1.wait()
                    pltpu.sync_copy(g1, acc.at[dst_v.at[j + 1]], add=True)

            plsc.subcore_barrier()
            pltpu.sync_copy(acc.at[pl.ds(sid * 624, 624)],
                            out_h.at[chunk].at[pl.ds(sid * 624, 624)])

            @pl.when(sid == 0)
            def _():
                pltpu.sync_copy(acc.at[pl.ds(9984, 16)],
                                out_h.at[chunk].at[pl.ds(9984, 16)])

            plsc.subcore_barrier()

    return k(vals, srcp, dstp)


def _scale_split_body(x_ref, dis_ref, o_ref):
    xd = x_ref[...] * dis_ref[...]
    o_ref[0] = xd[:, 0:128]
    o_ref[1] = xd[:, 128:256]


def _scale_split(x, dis):
    return pl.pallas_call(
        _scale_split_body,
        grid=(_N // _ROWS,),
        in_specs=[
            pl.BlockSpec((_ROWS, 256), lambda i: (i, 0)),
            pl.BlockSpec((_ROWS, 1), lambda i: (i, 0)),
        ],
        out_specs=pl.BlockSpec((2, _ROWS, 128), lambda i: (0, i, 0)),
        out_shape=jax.ShapeDtypeStruct((2, _N, 128), jnp.float32),
    )(x, dis)


def _mm_stats_body(agg_ref, xd_ref, dis_ref, w_ref, b_ref, y_ref, acc_ref):
    a = jnp.concatenate(
        [agg_ref[0] + xd_ref[0], agg_ref[1] + xd_ref[1]], axis=1)
    a = a * dis_ref[...]
    y = jnp.dot(a, w_ref[...], preferred_element_type=jnp.float32)
    y = y + b_ref[...][None, :]
    y_ref[...] = y

    @pl.when(pl.program_id(0) == 0)
    def _():
        acc_ref[...] = jnp.zeros_like(acc_ref)

    acc_ref[0:1, :] += jnp.sum(y, axis=0, keepdims=True)
    acc_ref[1:2, :] += jnp.sum(y * y, axis=0, keepdims=True)


def _mm_stats(agg, xd, dis, w, b):
    d = w.shape[1]
    return pl.pallas_call(
        _mm_stats_body,
        grid=(_N // _ROWS,),
        in_specs=[
            pl.BlockSpec((2, _ROWS, 128), lambda i: (0, i, 0)),
            pl.BlockSpec((2, _ROWS, 128), lambda i: (0, i, 0)),
            pl.BlockSpec((_ROWS, 1), lambda i: (i, 0)),
            pl.BlockSpec((256, d), lambda i: (0, 0)),
            pl.BlockSpec((d,), lambda i: (0,)),
        ],
        out_specs=[
            pl.BlockSpec((_ROWS, d), lambda i: (i, 0)),
            pl.BlockSpec((8, d), lambda i: (0, 0)),
        ],
        out_shape=[
            jax.ShapeDtypeStruct((_N, d), jnp.float32),
            jax.ShapeDtypeStruct((8, d), jnp.float32),
        ],
    )(agg, xd, dis, w, b)


def _bn_tanh_mm_body(y_ref, acc_ref, g_ref, be_ref, dis_ref, w_ref, p_ref):
    m = acc_ref[0:1, :] / _N
    v = acc_ref[1:2, :] / _N - m * m
    h = jnp.tanh((y_ref[...] - m) * lax.rsqrt(v + _EPS) * g_ref[...][None, :]
                 + be_ref[...][None, :])
    hd = h * dis_ref[...]
    p = jnp.dot(hd, w_ref[...], preferred_element_type=jnp.float32)
    for c in range(4):
        p_ref[c] = p[:, c * 128:(c + 1) * 128]


def _bn_tanh_mm(y, acc, g, be, dis, w):
    d = y.shape[1]
    do = w.shape[1]
    return pl.pallas_call(
        _bn_tanh_mm_body,
        grid=(_N // _ROWS,),
        in_specs=[
            pl.BlockSpec((_ROWS, d), lambda i: (i, 0)),
            pl.BlockSpec((8, d), lambda i: (0, 0)),
            pl.BlockSpec((d,), lambda i: (0,)),
            pl.BlockSpec((d,), lambda i: (0,)),
            pl.BlockSpec((_ROWS, 1), lambda i: (i, 0)),
            pl.BlockSpec((d, do), lambda i: (0, 0)),
        ],
        out_specs=pl.BlockSpec((4, _ROWS, 128), lambda i: (0, i, 0)),
        out_shape=jax.ShapeDtypeStruct((4, _N, 128), jnp.float32),
    )(y, acc, g, be, dis, w)


def _combine_stats_body(agg_ref, p_ref, dis_ref, b_ref, y_ref, acc_ref):
    y = jnp.concatenate([agg_ref[c] + p_ref[c] for c in range(4)], axis=1)
    y = y * dis_ref[...] + b_ref[...][None, :]
    y_ref[...] = y

    @pl.when(pl.program_id(0) == 0)
    def _():
        acc_ref[...] = jnp.zeros_like(acc_ref)

    acc_ref[0:1, :] += jnp.sum(y, axis=0, keepdims=True)
    acc_ref[1:2, :] += jnp.sum(y * y, axis=0, keepdims=True)


def _combine_stats(agg, p, dis, b):
    d = b.shape[0]
    return pl.pallas_call(
        _combine_stats_body,
        grid=(_N // _ROWS,),
        in_specs=[
            pl.BlockSpec((4, _ROWS, 128), lambda i: (0, i, 0)),
            pl.BlockSpec((4, _ROWS, 128), lambda i: (0, i, 0)),
            pl.BlockSpec((_ROWS, 1), lambda i: (i, 0)),
            pl.BlockSpec((d,), lambda i: (0,)),
        ],
        out_specs=[
            pl.BlockSpec((_ROWS, d), lambda i: (i, 0)),
            pl.BlockSpec((8, d), lambda i: (0, 0)),
        ],
        out_shape=[
            jax.ShapeDtypeStruct((_N, d), jnp.float32),
            jax.ShapeDtypeStruct((8, d), jnp.float32),
        ],
    )(agg, p, dis, b)


def _bn_tanh_pool_body(y_ref, acc_ref, g_ref, be_ref, batch_ref, s_ref,
                       cnt_ref):
    m = acc_ref[0:1, :] / _N
    v = acc_ref[1:2, :] / _N - m * m
    h = jnp.tanh((y_ref[...] - m) * lax.rsqrt(v + _EPS) * g_ref[...][None, :]
                 + be_ref[...][None, :])
    onehot = (batch_ref[...] ==
              jax.lax.broadcasted_iota(jnp.int32, (1, _G), 1)).astype(jnp.float32)

    @pl.when(pl.program_id(0) == 0)
    def _():
        s_ref[...] = jnp.zeros_like(s_ref)
        cnt_ref[...] = jnp.zeros_like(cnt_ref)

    dims = (((0,), (0,)), ((), ()))
    s_ref[...] += jax.lax.dot_general(onehot, h, dims,
                                      preferred_element_type=jnp.float32)
    cnt_ref[...] += jax.lax.dot_general(
        onehot, jnp.ones((onehot.shape[0], 128), jnp.float32), dims,
        preferred_element_type=jnp.float32)


def _bn_tanh_pool(y, acc, g, be, batch):
    d = y.shape[1]
    return pl.pallas_call(
        _bn_tanh_pool_body,
        grid=(_N // _ROWS,),
        in_specs=[
            pl.BlockSpec((_ROWS, d), lambda i: (i, 0)),
            pl.BlockSpec((8, d), lambda i: (0, 0)),
            pl.BlockSpec((d,), lambda i: (0,)),
            pl.BlockSpec((d,), lambda i: (0,)),
            pl.BlockSpec((_ROWS, 1), lambda i: (i, 0)),
        ],
        out_specs=[
            pl.BlockSpec((_G, d), lambda i: (0, 0)),
            pl.BlockSpec((_G, 128), lambda i: (0, 0)),
        ],
        out_shape=[
            jax.ShapeDtypeStruct((_G, d), jnp.float32),
            jax.ShapeDtypeStruct((_G, 128), jnp.float32),
        ],
    )(y, acc, g, be, batch)


def _head_body(s_ref, cnt_ref, wl_ref, bl_ref, gl_ref, bel_ref,
               wf_ref, bf_ref, gf_ref, bef_ref, o_ref):
    p = s_ref[...] / jnp.maximum(cnt_ref[:, 0:1], 1.0)
    y = jnp.dot(p, wl_ref[...], preferred_element_type=jnp.float32)
    y = y + bl_ref[...][None, :]
    m = jnp.mean(y, axis=0, keepdims=True)
    v = jnp.mean(y * y, axis=0, keepdims=True) - m * m
    z = jnp.tanh((y - m) * lax.rsqrt(v + _EPS) * gl_ref[...][None, :]
                 + bel_ref[...][None, :])
    o = jnp.dot(z, wf_ref[...], preferred_element_type=jnp.float32)
    o = o + bf_ref[...][None, :]
    m2 = jnp.mean(o, axis=0, keepdims=True)
    v2 = jnp.mean(o * o, axis=0, keepdims=True) - m2 * m2
    o = (o - m2) * lax.rsqrt(v2 + _EPS) * gf_ref[...][None, :] + bef_ref[...][None, :]
    o = o - jnp.max(o, axis=1, keepdims=True)
    o_ref[...] = o - jnp.log(jnp.sum(jnp.exp(o), axis=1, keepdims=True))


def _head(s, cnt, Wl, bl, gl, bel, Wf, bf, gf, bef):
    args = (s, cnt, Wl, bl, gl, bel, Wf, bf, gf, bef)
    return pl.pallas_call(
        _head_body,
        in_specs=[pl.BlockSpec(a.shape, (lambda *_, nd=a.ndim: (0,) * nd))
                  for a in args],
        out_specs=pl.BlockSpec((_G, Wf.shape[1]), lambda: (0, 0)),
        out_shape=jax.ShapeDtypeStruct((_G, Wf.shape[1]), jnp.float32),
    )(*args)


def kernel(x, edge_index, batch, W1, b1, g1, be1, W2, b2, g2, be2,
           Wl, bl, gl, bel, Wf, bf, gf, bef):
    src = edge_index[0]
    dst = edge_index[1]
    deg = jnp.ones((_N,), jnp.float32).at[dst].add(1.0)
    dis2 = lax.rsqrt(deg)[:, None]
    batch2 = batch[:, None]

    pad = _EPAD - _E
    srcp = jnp.concatenate([src, jnp.zeros((pad,), jnp.int32)]
                           ).reshape(_NSUB, _NBLK, _BLK)
    dstp = jnp.concatenate([dst, jnp.full((pad,), _N, jnp.int32)]
                           ).reshape(_NSUB, _NBLK, _BLK)

    xd = _scale_split(x, dis2)
    aggx = _seg_sum(xd, srcp, dstp)
    y1, acc1 = _mm_stats(aggx, xd, dis2, W1, b1)
    p2 = _bn_tanh_mm(y1, acc1, g1, be1, dis2, W2)
    agg2 = _seg_sum(p2, srcp, dstp)
    y2, acc2 = _combine_stats(agg2, p2, dis2, b2)
    s, cnt = _bn_tanh_pool(y2, acc2, g2, be2, batch2)
    return _head(s, cnt, Wl, bl, gl, bel, Wf, bf, gf, bef)


# speedup vs baseline: 6.5800x; 1.0405x over previous
import functools

import jax
import jax.numpy as jnp
from jax import lax
from jax.experimental import pallas as pl
from jax.experimental.pallas import tpu as pltpu
from jax.experimental.pallas import tpu_sc as plsc

_N = 10000
_E = 160000
_G = 64
_EPS = 1e-5
_ROWS = 1000

_NSUB = 16
_BLK = 128
_NBLK = 80
_EPAD = _NSUB * _NBLK * _BLK
_IGRP = 16
_ACCROWS = _N + 16


def _seg_sum(vals, srcp, dstp):
    C = vals.shape[0]
    cpc = C // 2
    mesh = plsc.VectorSubcoreMesh(core_axis_name="c", subcore_axis_name="s")

    @functools.partial(
        pl.kernel, mesh=mesh,
        out_type=jax.ShapeDtypeStruct((C, _N, 128), jnp.float32),
        scratch_types=[
            pltpu.VMEM((_IGRP, _BLK), jnp.int32),
            pltpu.VMEM((_IGRP, _BLK), jnp.int32),
            pltpu.VMEM((_BLK, 128), jnp.float32),
            pltpu.VMEM((_BLK, 128), jnp.float32),
            pltpu.VMEM_SHARED((_ACCROWS, 128), jnp.float32),
            pltpu.SemaphoreType.DMA,
            pltpu.SemaphoreType.DMA,
            pltpu.SemaphoreType.DMA,
            pltpu.SemaphoreType.DMA,
        ])
    def k(vals_h, src_h, dst_h, out_h, src_v, dst_v, g0, g1, acc,
          sem0, sem1, ssem0, ssem1):
        cid = lax.axis_index("c")
        sid = lax.axis_index("s")

        for t in range(cpc):
            chunk = cid * cpc + t
            vc = vals_h.at[chunk]

            @pl.loop(0, _BLK)
            def _(r):
                @pl.loop(0, 128, step=16)
                def _(l):
                    g0[r, pl.ds(l, 16)] = jnp.zeros((16,), jnp.float32)

            @pl.loop(0, 4)
            def _(z):
                pltpu.sync_copy(g0, acc.at[pl.ds(sid * 626 + z * 128, 128)])

            pltpu.sync_copy(g0.at[pl.ds(0, 114)],
                            acc.at[pl.ds(sid * 626 + 512, 114)])
            plsc.subcore_barrier()

            @pl.loop(0, _NBLK // _IGRP)
            def _(gi):
                pltpu.sync_copy(src_h.at[sid].at[pl.ds(gi * _IGRP, _IGRP)],
                                src_v)
                pltpu.sync_copy(dst_h.at[sid].at[pl.ds(gi * _IGRP, _IGRP)],
                                dst_v)

                pltpu.async_copy(vc.at[src_v.at[0]], g0, sem0)
                pltpu.async_copy(vc.at[src_v.at[1]], g1, sem1)

                @pl.loop(0, _IGRP - 2, step=2)
                def _(j):
                    pltpu.make_async_copy(vc.at[src_v.at[j]], g0, sem0).wait()
                    pltpu.async_copy(g0, acc.at[dst_v.at[j]], ssem0, add=True)
                    pltpu.make_async_copy(
                        vc.at[src_v.at[j + 1]], g1, sem1).wait()
                    pltpu.async_copy(g1, acc.at[dst_v.at[j + 1]], ssem1,
                                     add=True)
                    pltpu.make_async_copy(g0, acc.at[dst_v.at[j]],
                                          ssem0).wait()
                    pltpu.async_copy(vc.at[src_v.at[j + 2]], g0, sem0)
                    pltpu.make_async_copy(g1, acc.at[dst_v.at[j + 1]],
                                          ssem1).wait()
                    pltpu.async_copy(vc.at[src_v.at[j + 3]], g1, sem1)

                pltpu.make_async_copy(
                    vc.at[src_v.at[_IGRP - 2]], g0, sem0).wait()
                pltpu.sync_copy(g0, acc.at[dst_v.at[_IGRP - 2]], add=True)
                pltpu.make_async_copy(
                    vc.at[src_v.at[_IGRP - 1]], g1, sem1).wait()
                pltpu.sync_copy(g1, acc.at[dst_v.at[_IGRP - 1]], add=True)

            plsc.subcore_barrier()
            pltpu.sync_copy(acc.at[pl.ds(sid * 624, 624)],
                            out_h.at[chunk].at[pl.ds(sid * 624, 624)])

            @pl.when(sid == 0)
            def _():
                pltpu.sync_copy(acc.at[pl.ds(9984, 16)],
                                out_h.at[chunk].at[pl.ds(9984, 16)])

            plsc.subcore_barrier()

    return k(vals, srcp, dstp)


def _scale_split_body(x_ref, dis_ref, o_ref):
    xd = x_ref[...] * dis_ref[...]
    o_ref[0] = xd[:, 0:128]
    o_ref[1] = xd[:, 128:256]


def _scale_split(x, dis):
    return pl.pallas_call(
        _scale_split_body,
        grid=(_N // _ROWS,),
        in_specs=[
            pl.BlockSpec((_ROWS, 256), lambda i: (i, 0)),
            pl.BlockSpec((_ROWS, 1), lambda i: (i, 0)),
        ],
        out_specs=pl.BlockSpec((2, _ROWS, 128), lambda i: (0, i, 0)),
        out_shape=jax.ShapeDtypeStruct((2, _N, 128), jnp.float32),
    )(x, dis)


def _mm_stats_body(agg_ref, xd_ref, dis_ref, w_ref, b_ref, y_ref, acc_ref):
    a = jnp.concatenate(
        [agg_ref[0] + xd_ref[0], agg_ref[1] + xd_ref[1]], axis=1)
    a = a * dis_ref[...]
    y = jnp.dot(a, w_ref[...], preferred_element_type=jnp.float32)
    y = y + b_ref[...][None, :]
    y_ref[...] = y

    @pl.when(pl.program_id(0) == 0)
    def _():
        acc_ref[...] = jnp.zeros_like(acc_ref)

    acc_ref[0:1, :] += jnp.sum(y, axis=0, keepdims=True)
    acc_ref[1:2, :] += jnp.sum(y * y, axis=0, keepdims=True)


def _mm_stats(agg, xd, dis, w, b):
    d = w.shape[1]
    return pl.pallas_call(
        _mm_stats_body,
        grid=(_N // _ROWS,),
        in_specs=[
            pl.BlockSpec((2, _ROWS, 128), lambda i: (0, i, 0)),
            pl.BlockSpec((2, _ROWS, 128), lambda i: (0, i, 0)),
            pl.BlockSpec((_ROWS, 1), lambda i: (i, 0)),
            pl.BlockSpec((256, d), lambda i: (0, 0)),
            pl.BlockSpec((d,), lambda i: (0,)),
        ],
        out_specs=[
            pl.BlockSpec((_ROWS, d), lambda i: (i, 0)),
            pl.BlockSpec((8, d), lambda i: (0, 0)),
        ],
        out_shape=[
            jax.ShapeDtypeStruct((_N, d), jnp.float32),
            jax.ShapeDtypeStruct((8, d), jnp.float32),
        ],
    )(agg, xd, dis, w, b)


def _bn_tanh_mm_body(y_ref, acc_ref, g_ref, be_ref, dis_ref, w_ref, p_ref):
    m = acc_ref[0:1, :] / _N
    v = acc_ref[1:2, :] / _N - m * m
    h = jnp.tanh((y_ref[...] - m) * lax.rsqrt(v + _EPS) * g_ref[...][None, :]
                 + be_ref[...][None, :])
    hd = h * dis_ref[...]
    p = jnp.dot(hd, w_ref[...], preferred_element_type=jnp.float32)
    for c in range(4):
        p_ref[c] = p[:, c * 128:(c + 1) * 128]


def _bn_tanh_mm(y, acc, g, be, dis, w):
    d = y.shape[1]
    do = w.shape[1]
    return pl.pallas_call(
        _bn_tanh_mm_body,
        grid=(_N // _ROWS,),
        in_specs=[
            pl.BlockSpec((_ROWS, d), lambda i: (i, 0)),
            pl.BlockSpec((8, d), lambda i: (0, 0)),
            pl.BlockSpec((d,), lambda i: (0,)),
            pl.BlockSpec((d,), lambda i: (0,)),
            pl.BlockSpec((_ROWS, 1), lambda i: (i, 0)),
            pl.BlockSpec((d, do), lambda i: (0, 0)),
        ],
        out_specs=pl.BlockSpec((4, _ROWS, 128), lambda i: (0, i, 0)),
        out_shape=jax.ShapeDtypeStruct((4, _N, 128), jnp.float32),
    )(y, acc, g, be, dis, w)


def _combine_stats_body(agg_ref, p_ref, dis_ref, b_ref, y_ref, acc_ref):
    y = jnp.concatenate([agg_ref[c] + p_ref[c] for c in range(4)], axis=1)
    y = y * dis_ref[...] + b_ref[...][None, :]
    y_ref[...] = y

    @pl.when(pl.program_id(0) == 0)
    def _():
        acc_ref[...] = jnp.zeros_like(acc_ref)

    acc_ref[0:1, :] += jnp.sum(y, axis=0, keepdims=True)
    acc_ref[1:2, :] += jnp.sum(y * y, axis=0, keepdims=True)


def _combine_stats(agg, p, dis, b):
    d = b.shape[0]
    return pl.pallas_call(
        _combine_stats_body,
        grid=(_N // _ROWS,),
        in_specs=[
            pl.BlockSpec((4, _ROWS, 128), lambda i: (0, i, 0)),
            pl.BlockSpec((4, _ROWS, 128), lambda i: (0, i, 0)),
            pl.BlockSpec((_ROWS, 1), lambda i: (i, 0)),
            pl.BlockSpec((d,), lambda i: (0,)),
        ],
        out_specs=[
            pl.BlockSpec((_ROWS, d), lambda i: (i, 0)),
            pl.BlockSpec((8, d), lambda i: (0, 0)),
        ],
        out_shape=[
            jax.ShapeDtypeStruct((_N, d), jnp.float32),
            jax.ShapeDtypeStruct((8, d), jnp.float32),
        ],
    )(agg, p, dis, b)


def _bn_tanh_pool_body(y_ref, acc_ref, g_ref, be_ref, batch_ref, s_ref,
                       cnt_ref):
    m = acc_ref[0:1, :] / _N
    v = acc_ref[1:2, :] / _N - m * m
    h = jnp.tanh((y_ref[...] - m) * lax.rsqrt(v + _EPS) * g_ref[...][None, :]
                 + be_ref[...][None, :])
    onehot = (batch_ref[...] ==
              jax.lax.broadcasted_iota(jnp.int32, (1, _G), 1)).astype(jnp.float32)

    @pl.when(pl.program_id(0) == 0)
    def _():
        s_ref[...] = jnp.zeros_like(s_ref)
        cnt_ref[...] = jnp.zeros_like(cnt_ref)

    dims = (((0,), (0,)), ((), ()))
    s_ref[...] += jax.lax.dot_general(onehot, h, dims,
                                      preferred_element_type=jnp.float32)
    cnt_ref[...] += jax.lax.dot_general(
        onehot, jnp.ones((onehot.shape[0], 128), jnp.float32), dims,
        preferred_element_type=jnp.float32)


def _bn_tanh_pool(y, acc, g, be, batch):
    d = y.shape[1]
    return pl.pallas_call(
        _bn_tanh_pool_body,
        grid=(_N // _ROWS,),
        in_specs=[
            pl.BlockSpec((_ROWS, d), lambda i: (i, 0)),
            pl.BlockSpec((8, d), lambda i: (0, 0)),
            pl.BlockSpec((d,), lambda i: (0,)),
            pl.BlockSpec((d,), lambda i: (0,)),
            pl.BlockSpec((_ROWS, 1), lambda i: (i, 0)),
        ],
        out_specs=[
            pl.BlockSpec((_G, d), lambda i: (0, 0)),
            pl.BlockSpec((_G, 128), lambda i: (0, 0)),
        ],
        out_shape=[
            jax.ShapeDtypeStruct((_G, d), jnp.float32),
            jax.ShapeDtypeStruct((_G, 128), jnp.float32),
        ],
    )(y, acc, g, be, batch)


def _head_body(s_ref, cnt_ref, wl_ref, bl_ref, gl_ref, bel_ref,
               wf_ref, bf_ref, gf_ref, bef_ref, o_ref):
    p = s_ref[...] / jnp.maximum(cnt_ref[:, 0:1], 1.0)
    y = jnp.dot(p, wl_ref[...], preferred_element_type=jnp.float32)
    y = y + bl_ref[...][None, :]
    m = jnp.mean(y, axis=0, keepdims=True)
    v = jnp.mean(y * y, axis=0, keepdims=True) - m * m
    z = jnp.tanh((y - m) * lax.rsqrt(v + _EPS) * gl_ref[...][None, :]
                 + bel_ref[...][None, :])
    o = jnp.dot(z, wf_ref[...], preferred_element_type=jnp.float32)
    o = o + bf_ref[...][None, :]
    m2 = jnp.mean(o, axis=0, keepdims=True)
    v2 = jnp.mean(o * o, axis=0, keepdims=True) - m2 * m2
    o = (o - m2) * lax.rsqrt(v2 + _EPS) * gf_ref[...][None, :] + bef_ref[...][None, :]
    o = o - jnp.max(o, axis=1, keepdims=True)
    o_ref[...] = o - jnp.log(jnp.sum(jnp.exp(o), axis=1, keepdims=True))


def _head(s, cnt, Wl, bl, gl, bel, Wf, bf, gf, bef):
    args = (s, cnt, Wl, bl, gl, bel, Wf, bf, gf, bef)
    return pl.pallas_call(
        _head_body,
        in_specs=[pl.BlockSpec(a.shape, (lambda *_, nd=a.ndim: (0,) * nd))
                  for a in args],
        out_specs=pl.BlockSpec((_G, Wf.shape[1]), lambda: (0, 0)),
        out_shape=jax.ShapeDtypeStruct((_G, Wf.shape[1]), jnp.float32),
    )(*args)


def kernel(x, edge_index, batch, W1, b1, g1, be1, W2, b2, g2, be2,
           Wl, bl, gl, bel, Wf, bf, gf, bef):
    src = edge_index[0]
    dst = edge_index[1]
    deg = jnp.ones((_N,), jnp.float32).at[dst].add(1.0)
    dis2 = lax.rsqrt(deg)[:, None]
    batch2 = batch[:, None]

    pad = _EPAD - _E
    srcp = jnp.concatenate([src, jnp.zeros((pad,), jnp.int32)]
                           ).reshape(_NSUB, _NBLK, _BLK)
    dstp = jnp.concatenate([dst, jnp.full((pad,), _N, jnp.int32)]
                           ).reshape(_NSUB, _NBLK, _BLK)

    xd = _scale_split(x, dis2)
    aggx = _seg_sum(xd, srcp, dstp)
    y1, acc1 = _mm_stats(aggx, xd, dis2, W1, b1)
    p2 = _bn_tanh_mm(y1, acc1, g1, be1, dis2, W2)
    agg2 = _seg_sum(p2, srcp, dstp)
    y2, acc2 = _combine_stats(agg2, p2, dis2, b2)
    s, cnt = _bn_tanh_pool(y2, acc2, g2, be2, batch2)
    return _head(s, cnt, Wl, bl, gl, bel, Wf, bf, gf, bef)


# speedup vs baseline: 7.9777x; 1.2124x over previous
import dataclasses
import functools

import jax
import jax.numpy as jnp
from jax import lax
from jax.experimental import pallas as pl
from jax.experimental.pallas import tpu as pltpu
from jax.experimental.pallas import tpu_sc as plsc

_N = 10000
_E = 160000
_G = 64
_EPS = 1e-5
_ROWS = 1000

_NSUB = 16
_BLK = 128
_NBLK = 80
_EPAD = _NSUB * _NBLK * _BLK
_IGRP = 16
_ACCROWS = _N + 16
_DEGROWS = 10240


def _deg_sc(dstp):
    mesh = plsc.VectorSubcoreMesh(core_axis_name="c", subcore_axis_name="s")
    cp = pltpu.CompilerParams()
    if "needs_layout_passes" in pltpu.CompilerParams.__dataclass_fields__:
        cp = dataclasses.replace(cp, needs_layout_passes=False)

    @functools.partial(
        pl.kernel, mesh=mesh, compiler_params=cp,
        out_type=jax.ShapeDtypeStruct((2, _DEGROWS), jnp.float32),
        scratch_types=[
            pltpu.VMEM((_NBLK // 2, _BLK), jnp.int32),
            pltpu.VMEM((_DEGROWS,), jnp.float32),
            pltpu.VMEM((_NSUB, 640), jnp.float32),
            pltpu.VMEM_SHARED((_NSUB, _DEGROWS), jnp.float32),
        ])
    def k(dst_h, out_h, idx_v, part, red, shp):
        cid = lax.axis_index("c")
        sid = lax.axis_index("s")
        pltpu.sync_copy(dst_h.at[sid].at[pl.ds(cid * (_NBLK // 2),
                                               _NBLK // 2)], idx_v)

        @pl.loop(0, _DEGROWS, step=16)
        def _(r):
            part[pl.ds(r, 16)] = jnp.zeros((16,), jnp.float32)

        ones = jnp.ones((16,), jnp.float32)

        @pl.loop(0, _NBLK // 2)
        def _(j):
            @pl.loop(0, _BLK, step=16)
            def _(l):
                plsc.addupdate_scatter(part, [idx_v[j, pl.ds(l, 16)]], ones)

        pltpu.sync_copy(part, shp.at[sid])
        plsc.subcore_barrier()
        pltpu.sync_copy(shp.at[:, pl.ds(sid * 640, 640)], red)

        @pl.loop(0, 640, step=16)
        def _(l):
            acc16 = red[0, pl.ds(l, 16)]
            for t in range(1, _NSUB):
                acc16 = acc16 + red[t, pl.ds(l, 16)]
            red[0, pl.ds(l, 16)] = acc16

        pltpu.sync_copy(red.at[0], out_h.at[cid].at[pl.ds(sid * 640, 640)])

    return k(dstp)


def _seg_sum(vals, srcp, dstp):
    C = vals.shape[0]
    cpc = C // 2
    mesh = plsc.VectorSubcoreMesh(core_axis_name="c", subcore_axis_name="s")

    @functools.partial(
        pl.kernel, mesh=mesh,
        out_type=jax.ShapeDtypeStruct((C, _N, 128), jnp.float32),
        scratch_types=[
            pltpu.VMEM((_IGRP, _BLK), jnp.int32),
            pltpu.VMEM((_IGRP, _BLK), jnp.int32),
            pltpu.VMEM((_BLK, 128), jnp.float32),
            pltpu.VMEM((_BLK, 128), jnp.float32),
            pltpu.VMEM_SHARED((_ACCROWS, 128), jnp.float32),
            pltpu.SemaphoreType.DMA,
            pltpu.SemaphoreType.DMA,
            pltpu.SemaphoreType.DMA,
            pltpu.SemaphoreType.DMA,
        ])
    def k(vals_h, src_h, dst_h, out_h, src_v, dst_v, g0, g1, acc,
          sem0, sem1, ssem0, ssem1):
        cid = lax.axis_index("c")
        sid = lax.axis_index("s")

        for t in range(cpc):
            chunk = cid * cpc + t
            vc = vals_h.at[chunk]

            @pl.loop(0, _BLK)
            def _(r):
                @pl.loop(0, 128, step=16)
                def _(l):
                    g0[r, pl.ds(l, 16)] = jnp.zeros((16,), jnp.float32)

            @pl.loop(0, 4)
            def _(z):
                pltpu.sync_copy(g0, acc.at[pl.ds(sid * 626 + z * 128, 128)])

            pltpu.sync_copy(g0.at[pl.ds(0, 114)],
                            acc.at[pl.ds(sid * 626 + 512, 114)])
            plsc.subcore_barrier()

            @pl.loop(0, _NBLK // _IGRP)
            def _(gi):
                pltpu.sync_copy(src_h.at[sid].at[pl.ds(gi * _IGRP, _IGRP)],
                                src_v)
                pltpu.sync_copy(dst_h.at[sid].at[pl.ds(gi * _IGRP, _IGRP)],
                                dst_v)

                pltpu.async_copy(vc.at[src_v.at[0]], g0, sem0)
                pltpu.async_copy(vc.at[src_v.at[1]], g1, sem1)

                @pl.loop(0, _IGRP - 2, step=2)
                def _(j):
                    pltpu.make_async_copy(vc.at[src_v.at[j]], g0, sem0).wait()
                    pltpu.async_copy(g0, acc.at[dst_v.at[j]], ssem0, add=True)
                    pltpu.make_async_copy(
                        vc.at[src_v.at[j + 1]], g1, sem1).wait()
                    pltpu.async_copy(g1, acc.at[dst_v.at[j + 1]], ssem1,
                                     add=True)
                    pltpu.make_async_copy(g0, acc.at[dst_v.at[j]],
                                          ssem0).wait()
                    pltpu.async_copy(vc.at[src_v.at[j + 2]], g0, sem0)
                    pltpu.make_async_copy(g1, acc.at[dst_v.at[j + 1]],
                                          ssem1).wait()
                    pltpu.async_copy(vc.at[src_v.at[j + 3]], g1, sem1)

                pltpu.make_async_copy(
                    vc.at[src_v.at[_IGRP - 2]], g0, sem0).wait()
                pltpu.sync_copy(g0, acc.at[dst_v.at[_IGRP - 2]], add=True)
                pltpu.make_async_copy(
                    vc.at[src_v.at[_IGRP - 1]], g1, sem1).wait()
                pltpu.sync_copy(g1, acc.at[dst_v.at[_IGRP - 1]], add=True)

            plsc.subcore_barrier()
            pltpu.sync_copy(acc.at[pl.ds(sid * 624, 624)],
                            out_h.at[chunk].at[pl.ds(sid * 624, 624)])

            @pl.when(sid == 0)
            def _():
                pltpu.sync_copy(acc.at[pl.ds(9984, 16)],
                                out_h.at[chunk].at[pl.ds(9984, 16)])

            plsc.subcore_barrier()

    return k(vals, srcp, dstp)


def _scale_split_body(x_ref, dis_ref, o_ref):
    xd = x_ref[...] * dis_ref[...]
    o_ref[0] = xd[:, 0:128]
    o_ref[1] = xd[:, 128:256]


def _scale_split(x, dis):
    return pl.pallas_call(
        _scale_split_body,
        grid=(_N // _ROWS,),
        in_specs=[
            pl.BlockSpec((_ROWS, 256), lambda i: (i, 0)),
            pl.BlockSpec((_ROWS, 1), lambda i: (i, 0)),
        ],
        out_specs=pl.BlockSpec((2, _ROWS, 128), lambda i: (0, i, 0)),
        out_shape=jax.ShapeDtypeStruct((2, _N, 128), jnp.float32),
    )(x, dis)


def _mm_stats_body(agg_ref, xd_ref, dis_ref, w_ref, b_ref, y_ref, acc_ref):
    a = jnp.concatenate(
        [agg_ref[0] + xd_ref[0], agg_ref[1] + xd_ref[1]], axis=1)
    a = a * dis_ref[...]
    y = jnp.dot(a, w_ref[...], preferred_element_type=jnp.float32)
    y = y + b_ref[...][None, :]
    y_ref[...] = y

    @pl.when(pl.program_id(0) == 0)
    def _():
        acc_ref[...] = jnp.zeros_like(acc_ref)

    acc_ref[0:1, :] += jnp.sum(y, axis=0, keepdims=True)
    acc_ref[1:2, :] += jnp.sum(y * y, axis=0, keepdims=True)


def _mm_stats(agg, xd, dis, w, b):
    d = w.shape[1]
    return pl.pallas_call(
        _mm_stats_body,
        grid=(_N // _ROWS,),
        in_specs=[
            pl.BlockSpec((2, _ROWS, 128), lambda i: (0, i, 0)),
            pl.BlockSpec((2, _ROWS, 128), lambda i: (0, i, 0)),
            pl.BlockSpec((_ROWS, 1), lambda i: (i, 0)),
            pl.BlockSpec((256, d), lambda i: (0, 0)),
            pl.BlockSpec((d,), lambda i: (0,)),
        ],
        out_specs=[
            pl.BlockSpec((_ROWS, d), lambda i: (i, 0)),
            pl.BlockSpec((8, d), lambda i: (0, 0)),
        ],
        out_shape=[
            jax.ShapeDtypeStruct((_N, d), jnp.float32),
            jax.ShapeDtypeStruct((8, d), jnp.float32),
        ],
    )(agg, xd, dis, w, b)


def _bn_tanh_mm_body(y_ref, acc_ref, g_ref, be_ref, dis_ref, w_ref, p_ref):
    m = acc_ref[0:1, :] / _N
    v = acc_ref[1:2, :] / _N - m * m
    h = jnp.tanh((y_ref[...] - m) * lax.rsqrt(v + _EPS) * g_ref[...][None, :]
                 + be_ref[...][None, :])
    hd = h * dis_ref[...]
    p = jnp.dot(hd, w_ref[...], preferred_element_type=jnp.float32)
    for c in range(4):
        p_ref[c] = p[:, c * 128:(c + 1) * 128]


def _bn_tanh_mm(y, acc, g, be, dis, w):
    d = y.shape[1]
    do = w.shape[1]
    return pl.pallas_call(
        _bn_tanh_mm_body,
        grid=(_N // _ROWS,),
        in_specs=[
            pl.BlockSpec((_ROWS, d), lambda i: (i, 0)),
            pl.BlockSpec((8, d), lambda i: (0, 0)),
            pl.BlockSpec((d,), lambda i: (0,)),
            pl.BlockSpec((d,), lambda i: (0,)),
            pl.BlockSpec((_ROWS, 1), lambda i: (i, 0)),
            pl.BlockSpec((d, do), lambda i: (0, 0)),
        ],
        out_specs=pl.BlockSpec((4, _ROWS, 128), lambda i: (0, i, 0)),
        out_shape=jax.ShapeDtypeStruct((4, _N, 128), jnp.float32),
    )(y, acc, g, be, dis, w)


def _combine_stats_body(agg_ref, p_ref, dis_ref, b_ref, y_ref, acc_ref):
    y = jnp.concatenate([agg_ref[c] + p_ref[c] for c in range(4)], axis=1)
    y = y * dis_ref[...] + b_ref[...][None, :]
    y_ref[...] = y

    @pl.when(pl.program_id(0) == 0)
    def _():
        acc_ref[...] = jnp.zeros_like(acc_ref)

    acc_ref[0:1, :] += jnp.sum(y, axis=0, keepdims=True)
    acc_ref[1:2, :] += jnp.sum(y * y, axis=0, keepdims=True)


def _combine_stats(agg, p, dis, b):
    d = b.shape[0]
    return pl.pallas_call(
        _combine_stats_body,
        grid=(_N // _ROWS,),
        in_specs=[
            pl.BlockSpec((4, _ROWS, 128), lambda i: (0, i, 0)),
            pl.BlockSpec((4, _ROWS, 128), lambda i: (0, i, 0)),
            pl.BlockSpec((_ROWS, 1), lambda i: (i, 0)),
            pl.BlockSpec((d,), lambda i: (0,)),
        ],
        out_specs=[
            pl.BlockSpec((_ROWS, d), lambda i: (i, 0)),
            pl.BlockSpec((8, d), lambda i: (0, 0)),
        ],
        out_shape=[
            jax.ShapeDtypeStruct((_N, d), jnp.float32),
            jax.ShapeDtypeStruct((8, d), jnp.float32),
        ],
    )(agg, p, dis, b)


def _bn_tanh_pool_body(y_ref, acc_ref, g_ref, be_ref, batch_ref, s_ref,
                       cnt_ref):
    m = acc_ref[0:1, :] / _N
    v = acc_ref[1:2, :] / _N - m * m
    h = jnp.tanh((y_ref[...] - m) * lax.rsqrt(v + _EPS) * g_ref[...][None, :]
                 + be_ref[...][None, :])
    onehot = (batch_ref[...] ==
              jax.lax.broadcasted_iota(jnp.int32, (1, _G), 1)).astype(jnp.float32)

    @pl.when(pl.program_id(0) == 0)
    def _():
        s_ref[...] = jnp.zeros_like(s_ref)
        cnt_ref[...] = jnp.zeros_like(cnt_ref)

    dims = (((0,), (0,)), ((), ()))
    s_ref[...] += jax.lax.dot_general(onehot, h, dims,
                                      preferred_element_type=jnp.float32)
    cnt_ref[...] += jax.lax.dot_general(
        onehot, jnp.ones((onehot.shape[0], 128), jnp.float32), dims,
        preferred_element_type=jnp.float32)


def _bn_tanh_pool(y, acc, g, be, batch):
    d = y.shape[1]
    return pl.pallas_call(
        _bn_tanh_pool_body,
        grid=(_N // _ROWS,),
        in_specs=[
            pl.BlockSpec((_ROWS, d), lambda i: (i, 0)),
            pl.BlockSpec((8, d), lambda i: (0, 0)),
            pl.BlockSpec((d,), lambda i: (0,)),
            pl.BlockSpec((d,), lambda i: (0,)),
            pl.BlockSpec((_ROWS, 1), lambda i: (i, 0)),
        ],
        out_specs=[
            pl.BlockSpec((_G, d), lambda i: (0, 0)),
            pl.BlockSpec((_G, 128), lambda i: (0, 0)),
        ],
        out_shape=[
            jax.ShapeDtypeStruct((_G, d), jnp.float32),
            jax.ShapeDtypeStruct((_G, 128), jnp.float32),
        ],
    )(y, acc, g, be, batch)


def _head_body(s_ref, cnt_ref, wl_ref, bl_ref, gl_ref, bel_ref,
               wf_ref, bf_ref, gf_ref, bef_ref, o_ref):
    p = s_ref[...] / jnp.maximum(cnt_ref[:, 0:1], 1.0)
    y = jnp.dot(p, wl_ref[...], preferred_element_type=jnp.float32)
    y = y + bl_ref[...][None, :]
    m = jnp.mean(y, axis=0, keepdims=True)
    v = jnp.mean(y * y, axis=0, keepdims=True) - m * m
    z = jnp.tanh((y - m) * lax.rsqrt(v + _EPS) * gl_ref[...][None, :]
                 + bel_ref[...][None, :])
    o = jnp.dot(z, wf_ref[...], preferred_element_type=jnp.float32)
    o = o + bf_ref[...][None, :]
    m2 = jnp.mean(o, axis=0, keepdims=True)
    v2 = jnp.mean(o * o, axis=0, keepdims=True) - m2 * m2
    o = (o - m2) * lax.rsqrt(v2 + _EPS) * gf_ref[...][None, :] + bef_ref[...][None, :]
    o = o - jnp.max(o, axis=1, keepdims=True)
    o_ref[...] = o - jnp.log(jnp.sum(jnp.exp(o), axis=1, keepdims=True))


def _head(s, cnt, Wl, bl, gl, bel, Wf, bf, gf, bef):
    args = (s, cnt, Wl, bl, gl, bel, Wf, bf, gf, bef)
    return pl.pallas_call(
        _head_body,
        in_specs=[pl.BlockSpec(a.shape, (lambda *_, nd=a.ndim: (0,) * nd))
                  for a in args],
        out_specs=pl.BlockSpec((_G, Wf.shape[1]), lambda: (0, 0)),
        out_shape=jax.ShapeDtypeStruct((_G, Wf.shape[1]), jnp.float32),
    )(*args)


def kernel(x, edge_index, batch, W1, b1, g1, be1, W2, b2, g2, be2,
           Wl, bl, gl, bel, Wf, bf, gf, bef):
    src = edge_index[0]
    dst = edge_index[1]
    batch2 = batch[:, None]

    pad = _EPAD - _E
    srcp = jnp.concatenate([src, jnp.zeros((pad,), jnp.int32)]
                           ).reshape(_NSUB, _NBLK, _BLK)
    dstp = jnp.concatenate([dst, jnp.full((pad,), _N, jnp.int32)]
                           ).reshape(_NSUB, _NBLK, _BLK)

    degp = _deg_sc(dstp)
    deg = 1.0 + degp[0, :_N] + degp[1, :_N]
    dis2 = lax.rsqrt(deg)[:, None]

    xd = _scale_split(x, dis2)
    aggx = _seg_sum(xd, srcp, dstp)
    y1, acc1 = _mm_stats(aggx, xd, dis2, W1, b1)
    p2 = _bn_tanh_mm(y1, acc1, g1, be1, dis2, W2)
    agg2 = _seg_sum(p2, srcp, dstp)
    y2, acc2 = _combine_stats(agg2, p2, dis2, b2)
    s, cnt = _bn_tanh_pool(y2, acc2, g2, be2, batch2)
    return _head(s, cnt, Wl, bl, gl, bel, Wf, bf, gf, bef)


# speedup vs baseline: 8.1160x; 1.0173x over previous
import dataclasses
import functools

import jax
import jax.numpy as jnp
from jax import lax
from jax.experimental import pallas as pl
from jax.experimental.pallas import tpu as pltpu
from jax.experimental.pallas import tpu_sc as plsc

_N = 10000
_E = 160000
_G = 64
_EPS = 1e-5
_ROWS = 1000

_NSUB = 16
_BLK = 128
_NBLK = 80
_EPAD = _NSUB * _NBLK * _BLK
_IGRP = 16
_ACCROWS = _N + 16
_DEGROWS = 10240


def _deg_sc(dstp):
    mesh = plsc.VectorSubcoreMesh(core_axis_name="c", subcore_axis_name="s")
    cp = pltpu.CompilerParams()
    if "needs_layout_passes" in pltpu.CompilerParams.__dataclass_fields__:
        cp = dataclasses.replace(cp, needs_layout_passes=False)

    @functools.partial(
        pl.kernel, mesh=mesh, compiler_params=cp,
        out_type=jax.ShapeDtypeStruct((2, _DEGROWS), jnp.float32),
        scratch_types=[
            pltpu.VMEM((_NBLK // 2, _BLK), jnp.int32),
            pltpu.VMEM((_DEGROWS,), jnp.float32),
            pltpu.VMEM((_NSUB, 640), jnp.float32),
            pltpu.VMEM_SHARED((_NSUB, _DEGROWS), jnp.float32),
        ])
    def k(dst_h, out_h, idx_v, part, red, shp):
        cid = lax.axis_index("c")
        sid = lax.axis_index("s")
        pltpu.sync_copy(dst_h.at[sid].at[pl.ds(cid * (_NBLK // 2),
                                               _NBLK // 2)], idx_v)

        @pl.loop(0, _DEGROWS, step=16)
        def _(r):
            part[pl.ds(r, 16)] = jnp.zeros((16,), jnp.float32)

        ones = jnp.ones((16,), jnp.float32)

        @pl.loop(0, _NBLK // 2)
        def _(j):
            @pl.loop(0, _BLK, step=16)
            def _(l):
                plsc.addupdate_scatter(part, [idx_v[j, pl.ds(l, 16)]], ones)

        pltpu.sync_copy(part, shp.at[sid])
        plsc.subcore_barrier()
        pltpu.sync_copy(shp.at[:, pl.ds(sid * 640, 640)], red)

        @pl.loop(0, 640, step=16)
        def _(l):
            acc16 = red[0, pl.ds(l, 16)]
            for t in range(1, _NSUB):
                acc16 = acc16 + red[t, pl.ds(l, 16)]
            red[0, pl.ds(l, 16)] = acc16

        pltpu.sync_copy(red.at[0], out_h.at[cid].at[pl.ds(sid * 640, 640)])

    return k(dstp)


def _seg_sum(vals, srcp, dstp):
    C = vals.shape[0]
    cpc = C // 2
    mesh = plsc.VectorSubcoreMesh(core_axis_name="c", subcore_axis_name="s")

    @functools.partial(
        pl.kernel, mesh=mesh,
        out_type=jax.ShapeDtypeStruct((C, _N, 128), jnp.float32),
        scratch_types=[
            pltpu.VMEM((_NBLK, _BLK), jnp.int32),
            pltpu.VMEM((_NBLK // 2, _BLK), jnp.int32),
            pltpu.VMEM((_BLK, 128), jnp.float32),
            pltpu.VMEM((_BLK, 128), jnp.float32),
            pltpu.VMEM_SHARED((_ACCROWS, 128), jnp.float32),
            pltpu.SemaphoreType.DMA,
            pltpu.SemaphoreType.DMA,
            pltpu.SemaphoreType.DMA,
            pltpu.SemaphoreType.DMA,
        ])
    def k(vals_h, src_h, dst_h, out_h, src_v, dst_v, g0, g1, acc,
          sem0, sem1, ssem0, ssem1):
        cid = lax.axis_index("c")
        sid = lax.axis_index("s")

        pltpu.sync_copy(src_h.at[sid], src_v)
        half = _NBLK // 2

        for t in range(cpc):
            chunk = cid * cpc + t
            vc = vals_h.at[chunk]

            @pl.loop(0, _BLK)
            def _(r):
                @pl.loop(0, 128, step=16)
                def _(l):
                    g0[r, pl.ds(l, 16)] = jnp.zeros((16,), jnp.float32)

            @pl.loop(0, 4)
            def _(z):
                pltpu.sync_copy(g0, acc.at[pl.ds(sid * 626 + z * 128, 128)])

            pltpu.sync_copy(g0.at[pl.ds(0, 114)],
                            acc.at[pl.ds(sid * 626 + 512, 114)])
            plsc.subcore_barrier()

            for gi in range(2):
                base = gi * half
                pltpu.sync_copy(dst_h.at[sid].at[pl.ds(base, half)], dst_v)
                pltpu.async_copy(vc.at[src_v.at[base]], g0, sem0)
                pltpu.async_copy(vc.at[src_v.at[base + 1]], g1, sem1)

                @pl.loop(0, half - 2, step=2)
                def _(j):
                    pltpu.make_async_copy(
                        vc.at[src_v.at[base + j]], g0, sem0).wait()
                    pltpu.async_copy(g0, acc.at[dst_v.at[j]], ssem0,
                                     add=True)
                    pltpu.make_async_copy(
                        vc.at[src_v.at[base + j + 1]], g1, sem1).wait()
                    pltpu.async_copy(g1, acc.at[dst_v.at[j + 1]], ssem1,
                                     add=True)
                    pltpu.make_async_copy(g0, acc.at[dst_v.at[j]],
                                          ssem0).wait()
                    pltpu.async_copy(vc.at[src_v.at[base + j + 2]], g0, sem0)
                    pltpu.make_async_copy(g1, acc.at[dst_v.at[j + 1]],
                                          ssem1).wait()
                    pltpu.async_copy(vc.at[src_v.at[base + j + 3]], g1, sem1)

                pltpu.make_async_copy(
                    vc.at[src_v.at[base + half - 2]], g0, sem0).wait()
                pltpu.sync_copy(g0, acc.at[dst_v.at[half - 2]], add=True)
                pltpu.make_async_copy(
                    vc.at[src_v.at[base + half - 1]], g1, sem1).wait()
                pltpu.sync_copy(g1, acc.at[dst_v.at[half - 1]], add=True)

            plsc.subcore_barrier()
            pltpu.sync_copy(acc.at[pl.ds(sid * 624, 624)],
                            out_h.at[chunk].at[pl.ds(sid * 624, 624)])

            @pl.when(sid == 0)
            def _():
                pltpu.sync_copy(acc.at[pl.ds(9984, 16)],
                                out_h.at[chunk].at[pl.ds(9984, 16)])

            plsc.subcore_barrier()

    return k(vals, srcp, dstp)


def _scale_split_body(x_ref, dis_ref, o_ref):
    xd = x_ref[...] * dis_ref[...]
    o_ref[0] = xd[:, 0:128]
    o_ref[1] = xd[:, 128:256]


def _scale_split(x, dis):
    return pl.pallas_call(
        _scale_split_body,
        grid=(_N // _ROWS,),
        in_specs=[
            pl.BlockSpec((_ROWS, 256), lambda i: (i, 0)),
            pl.BlockSpec((_ROWS, 1), lambda i: (i, 0)),
        ],
        out_specs=pl.BlockSpec((2, _ROWS, 128), lambda i: (0, i, 0)),
        out_shape=jax.ShapeDtypeStruct((2, _N, 128), jnp.float32),
    )(x, dis)


def _mm_stats_body(agg_ref, xd_ref, dis_ref, w_ref, b_ref, y_ref, acc_ref):
    a = jnp.concatenate(
        [agg_ref[0] + xd_ref[0], agg_ref[1] + xd_ref[1]], axis=1)
    a = a * dis_ref[...]
    y = jnp.dot(a, w_ref[...], preferred_element_type=jnp.float32)
    y = y + b_ref[...][None, :]
    y_ref[...] = y

    @pl.when(pl.program_id(0) == 0)
    def _():
        acc_ref[...] = jnp.zeros_like(acc_ref)

    acc_ref[0:1, :] += jnp.sum(y, axis=0, keepdims=True)
    acc_ref[1:2, :] += jnp.sum(y * y, axis=0, keepdims=True)


def _mm_stats(agg, xd, dis, w, b):
    d = w.shape[1]
    return pl.pallas_call(
        _mm_stats_body,
        grid=(_N // _ROWS,),
        in_specs=[
            pl.BlockSpec((2, _ROWS, 128), lambda i: (0, i, 0)),
            pl.BlockSpec((2, _ROWS, 128), lambda i: (0, i, 0)),
            pl.BlockSpec((_ROWS, 1), lambda i: (i, 0)),
            pl.BlockSpec((256, d), lambda i: (0, 0)),
            pl.BlockSpec((d,), lambda i: (0,)),
        ],
        out_specs=[
            pl.BlockSpec((_ROWS, d), lambda i: (i, 0)),
            pl.BlockSpec((8, d), lambda i: (0, 0)),
        ],
        out_shape=[
            jax.ShapeDtypeStruct((_N, d), jnp.float32),
            jax.ShapeDtypeStruct((8, d), jnp.float32),
        ],
    )(agg, xd, dis, w, b)


def _bn_tanh_mm_body(y_ref, acc_ref, g_ref, be_ref, dis_ref, w_ref, p_ref):
    m = acc_ref[0:1, :] / _N
    v = acc_ref[1:2, :] / _N - m * m
    h = jnp.tanh((y_ref[...] - m) * lax.rsqrt(v + _EPS) * g_ref[...][None, :]
                 + be_ref[...][None, :])
    hd = h * dis_ref[...]
    p = jnp.dot(hd, w_ref[...], preferred_element_type=jnp.float32)
    for c in range(4):
        p_ref[c] = p[:, c * 128:(c + 1) * 128]


def _bn_tanh_mm(y, acc, g, be, dis, w):
    d = y.shape[1]
    do = w.shape[1]
    return pl.pallas_call(
        _bn_tanh_mm_body,
        grid=(_N // _ROWS,),
        in_specs=[
            pl.BlockSpec((_ROWS, d), lambda i: (i, 0)),
            pl.BlockSpec((8, d), lambda i: (0, 0)),
            pl.BlockSpec((d,), lambda i: (0,)),
            pl.BlockSpec((d,), lambda i: (0,)),
            pl.BlockSpec((_ROWS, 1), lambda i: (i, 0)),
            pl.BlockSpec((d, do), lambda i: (0, 0)),
        ],
        out_specs=pl.BlockSpec((4, _ROWS, 128), lambda i: (0, i, 0)),
        out_shape=jax.ShapeDtypeStruct((4, _N, 128), jnp.float32),
    )(y, acc, g, be, dis, w)


def _combine_stats_body(agg_ref, p_ref, dis_ref, b_ref, y_ref, acc_ref):
    y = jnp.concatenate([agg_ref[c] + p_ref[c] for c in range(4)], axis=1)
    y = y * dis_ref[...] + b_ref[...][None, :]
    y_ref[...] = y

    @pl.when(pl.program_id(0) == 0)
    def _():
        acc_ref[...] = jnp.zeros_like(acc_ref)

    acc_ref[0:1, :] += jnp.sum(y, axis=0, keepdims=True)
    acc_ref[1:2, :] += jnp.sum(y * y, axis=0, keepdims=True)


def _combine_stats(agg, p, dis, b):
    d = b.shape[0]
    return pl.pallas_call(
        _combine_stats_body,
        grid=(_N // _ROWS,),
        in_specs=[
            pl.BlockSpec((4, _ROWS, 128), lambda i: (0, i, 0)),
            pl.BlockSpec((4, _ROWS, 128), lambda i: (0, i, 0)),
            pl.BlockSpec((_ROWS, 1), lambda i: (i, 0)),
            pl.BlockSpec((d,), lambda i: (0,)),
        ],
        out_specs=[
            pl.BlockSpec((_ROWS, d), lambda i: (i, 0)),
            pl.BlockSpec((8, d), lambda i: (0, 0)),
        ],
        out_shape=[
            jax.ShapeDtypeStruct((_N, d), jnp.float32),
            jax.ShapeDtypeStruct((8, d), jnp.float32),
        ],
    )(agg, p, dis, b)


def _bn_tanh_pool_body(y_ref, acc_ref, g_ref, be_ref, batch_ref, s_ref,
                       cnt_ref):
    m = acc_ref[0:1, :] / _N
    v = acc_ref[1:2, :] / _N - m * m
    h = jnp.tanh((y_ref[...] - m) * lax.rsqrt(v + _EPS) * g_ref[...][None, :]
                 + be_ref[...][None, :])
    onehot = (batch_ref[...] ==
              jax.lax.broadcasted_iota(jnp.int32, (1, _G), 1)).astype(jnp.float32)

    @pl.when(pl.program_id(0) == 0)
    def _():
        s_ref[...] = jnp.zeros_like(s_ref)
        cnt_ref[...] = jnp.zeros_like(cnt_ref)

    dims = (((0,), (0,)), ((), ()))
    s_ref[...] += jax.lax.dot_general(onehot, h, dims,
                                      preferred_element_type=jnp.float32)
    cnt_ref[...] += jax.lax.dot_general(
        onehot, jnp.ones((onehot.shape[0], 128), jnp.float32), dims,
        preferred_element_type=jnp.float32)


def _bn_tanh_pool(y, acc, g, be, batch):
    d = y.shape[1]
    return pl.pallas_call(
        _bn_tanh_pool_body,
        grid=(_N // _ROWS,),
        in_specs=[
            pl.BlockSpec((_ROWS, d), lambda i: (i, 0)),
            pl.BlockSpec((8, d), lambda i: (0, 0)),
            pl.BlockSpec((d,), lambda i: (0,)),
            pl.BlockSpec((d,), lambda i: (0,)),
            pl.BlockSpec((_ROWS, 1), lambda i: (i, 0)),
        ],
        out_specs=[
            pl.BlockSpec((_G, d), lambda i: (0, 0)),
            pl.BlockSpec((_G, 128), lambda i: (0, 0)),
        ],
        out_shape=[
            jax.ShapeDtypeStruct((_G, d), jnp.float32),
            jax.ShapeDtypeStruct((_G, 128), jnp.float32),
        ],
    )(y, acc, g, be, batch)


def _head_body(s_ref, cnt_ref, wl_ref, bl_ref, gl_ref, bel_ref,
               wf_ref, bf_ref, gf_ref, bef_ref, o_ref):
    p = s_ref[...] / jnp.maximum(cnt_ref[:, 0:1], 1.0)
    y = jnp.dot(p, wl_ref[...], preferred_element_type=jnp.float32)
    y = y + bl_ref[...][None, :]
    m = jnp.mean(y, axis=0, keepdims=True)
    v = jnp.mean(y * y, axis=0, keepdims=True) - m * m
    z = jnp.tanh((y - m) * lax.rsqrt(v + _EPS) * gl_ref[...][None, :]
                 + bel_ref[...][None, :])
    o = jnp.dot(z, wf_ref[...], preferred_element_type=jnp.float32)
    o = o + bf_ref[...][None, :]
    m2 = jnp.mean(o, axis=0, keepdims=True)
    v2 = jnp.mean(o * o, axis=0, keepdims=True) - m2 * m2
    o = (o - m2) * lax.rsqrt(v2 + _EPS) * gf_ref[...][None, :] + bef_ref[...][None, :]
    o = o - jnp.max(o, axis=1, keepdims=True)
    o_ref[...] = o - jnp.log(jnp.sum(jnp.exp(o), axis=1, keepdims=True))


def _head(s, cnt, Wl, bl, gl, bel, Wf, bf, gf, bef):
    args = (s, cnt, Wl, bl, gl, bel, Wf, bf, gf, bef)
    return pl.pallas_call(
        _head_body,
        in_specs=[pl.BlockSpec(a.shape, (lambda *_, nd=a.ndim: (0,) * nd))
                  for a in args],
        out_specs=pl.BlockSpec((_G, Wf.shape[1]), lambda: (0, 0)),
        out_shape=jax.ShapeDtypeStruct((_G, Wf.shape[1]), jnp.float32),
    )(*args)


def kernel(x, edge_index, batch, W1, b1, g1, be1, W2, b2, g2, be2,
           Wl, bl, gl, bel, Wf, bf, gf, bef):
    src = edge_index[0]
    dst = edge_index[1]
    batch2 = batch[:, None]

    pad = _EPAD - _E
    srcp = jnp.concatenate([src, jnp.zeros((pad,), jnp.int32)]
                           ).reshape(_NSUB, _NBLK, _BLK)
    dstp = jnp.concatenate([dst, jnp.full((pad,), _N, jnp.int32)]
                           ).reshape(_NSUB, _NBLK, _BLK)

    degp = _deg_sc(dstp)
    deg = 1.0 + degp[0, :_N] + degp[1, :_N]
    dis2 = lax.rsqrt(deg)[:, None]

    xd = _scale_split(x, dis2)
    aggx = _seg_sum(xd, srcp, dstp)
    y1, acc1 = _mm_stats(aggx, xd, dis2, W1, b1)
    p2 = _bn_tanh_mm(y1, acc1, g1, be1, dis2, W2)
    agg2 = _seg_sum(p2, srcp, dstp)
    y2, acc2 = _combine_stats(agg2, p2, dis2, b2)
    s, cnt = _bn_tanh_pool(y2, acc2, g2, be2, batch2)
    return _head(s, cnt, Wl, bl, gl, bel, Wf, bf, gf, bef)


# speedup vs baseline: 8.1214x; 1.0007x over previous
import functools

import jax
import jax.numpy as jnp
from jax import lax
from jax.experimental import pallas as pl
from jax.experimental.pallas import tpu as pltpu
from jax.experimental.pallas import tpu_sc as plsc

_N = 10000
_E = 160000
_G = 64
_EPS = 1e-5
_ROWS = 1000

_NSUB = 16
_BLK = 128
_NBLK = 80
_EPAD = _NSUB * _NBLK * _BLK
_ACCROWS = _N + 16
_DEGROWS = 10240


def _deg_sc(dstp):
    mesh = plsc.VectorSubcoreMesh(core_axis_name="c", subcore_axis_name="s")

    @functools.partial(
        pl.kernel, mesh=mesh,
        compiler_params=pltpu.CompilerParams(needs_layout_passes=False),
        out_type=jax.ShapeDtypeStruct((2, _DEGROWS), jnp.float32),
        scratch_types=[
            pltpu.VMEM((_NBLK // 2, _BLK), jnp.int32),
            pltpu.VMEM((_DEGROWS,), jnp.float32),
            pltpu.VMEM((_NSUB, 640), jnp.float32),
            pltpu.VMEM_SHARED((_NSUB, _DEGROWS), jnp.float32),
        ])
    def k(dst_h, out_h, idx_v, part, red, shp):
        cid = lax.axis_index("c")
        sid = lax.axis_index("s")
        pltpu.sync_copy(dst_h.at[sid].at[pl.ds(cid * (_NBLK // 2),
                                               _NBLK // 2)], idx_v)

        @pl.loop(0, _DEGROWS, step=16)
        def _(r):
            part[pl.ds(r, 16)] = jnp.zeros((16,), jnp.float32)

        ones = jnp.ones((16,), jnp.float32)

        @pl.loop(0, _NBLK // 2)
        def _(j):
            @pl.loop(0, _BLK, step=16)
            def _(l):
                plsc.addupdate_scatter(part, [idx_v[j, pl.ds(l, 16)]], ones)

        pltpu.sync_copy(part, shp.at[sid])
        plsc.subcore_barrier()
        pltpu.sync_copy(shp.at[:, pl.ds(sid * 640, 640)], red)

        @pl.loop(0, 640, step=16)
        def _(l):
            acc16 = red[0, pl.ds(l, 16)]
            for t in range(1, _NSUB):
                acc16 = acc16 + red[t, pl.ds(l, 16)]
            red[0, pl.ds(l, 16)] = acc16

        pltpu.sync_copy(red.at[0], out_h.at[cid].at[pl.ds(sid * 640, 640)])

    return k(dstp)


def _seg_sum(vals, srcp, dstp):
    C = vals.shape[0]
    cpc = C // 2
    mesh = plsc.VectorSubcoreMesh(core_axis_name="c", subcore_axis_name="s")

    @functools.partial(
        pl.kernel, mesh=mesh,
        out_type=jax.ShapeDtypeStruct((C, _N, 128), jnp.float32),
        scratch_types=[
            pltpu.VMEM((_NBLK, _BLK), jnp.int32),
            pltpu.VMEM((_NBLK // 2, _BLK), jnp.int32),
            pltpu.VMEM((_BLK, 128), jnp.float32),
            pltpu.VMEM((_BLK, 128), jnp.float32),
            pltpu.VMEM_SHARED((_ACCROWS, 128), jnp.float32),
            pltpu.SemaphoreType.DMA,
            pltpu.SemaphoreType.DMA,
            pltpu.SemaphoreType.DMA,
            pltpu.SemaphoreType.DMA,
        ])
    def k(vals_h, src_h, dst_h, out_h, src_v, dst_v, g0, g1, acc,
          sem0, sem1, ssem0, ssem1):
        cid = lax.axis_index("c")
        sid = lax.axis_index("s")

        pltpu.sync_copy(src_h.at[sid], src_v)
        half = _NBLK // 2

        for t in range(cpc):
            chunk = cid * cpc + t
            vc = vals_h.at[chunk]

            @pl.loop(0, _BLK)
            def _(r):
                @pl.loop(0, 128, step=16)
                def _(l):
                    g0[r, pl.ds(l, 16)] = jnp.zeros((16,), jnp.float32)

            @pl.loop(0, 4)
            def _(z):
                pltpu.sync_copy(g0, acc.at[pl.ds(sid * 626 + z * 128, 128)])

            pltpu.sync_copy(g0.at[pl.ds(0, 114)],
                            acc.at[pl.ds(sid * 626 + 512, 114)])
            plsc.subcore_barrier()

            for gi in range(2):
                base = gi * half
                pltpu.sync_copy(dst_h.at[sid].at[pl.ds(base, half)], dst_v)
                pltpu.async_copy(vc.at[src_v.at[base]], g0, sem0)
                pltpu.async_copy(vc.at[src_v.at[base + 1]], g1, sem1)

                @pl.loop(0, half - 2, step=2)
                def _(j):
                    pltpu.make_async_copy(
                        vc.at[src_v.at[base + j]], g0, sem0).wait()
                    pltpu.async_copy(g0, acc.at[dst_v.at[j]], ssem0,
                                     add=True)
                    pltpu.make_async_copy(
                        vc.at[src_v.at[base + j + 1]], g1, sem1).wait()
                    pltpu.async_copy(g1, acc.at[dst_v.at[j + 1]], ssem1,
                                     add=True)
                    pltpu.make_async_copy(g0, acc.at[dst_v.at[j]],
                                          ssem0).wait()
                    pltpu.async_copy(vc.at[src_v.at[base + j + 2]], g0, sem0)
                    pltpu.make_async_copy(g1, acc.at[dst_v.at[j + 1]],
                                          ssem1).wait()
                    pltpu.async_copy(vc.at[src_v.at[base + j + 3]], g1, sem1)

                pltpu.make_async_copy(
                    vc.at[src_v.at[base + half - 2]], g0, sem0).wait()
                pltpu.sync_copy(g0, acc.at[dst_v.at[half - 2]], add=True)
                pltpu.make_async_copy(
                    vc.at[src_v.at[base + half - 1]], g1, sem1).wait()
                pltpu.sync_copy(g1, acc.at[dst_v.at[half - 1]], add=True)

            plsc.subcore_barrier()
            pltpu.sync_copy(acc.at[pl.ds(sid * 624, 624)],
                            out_h.at[chunk].at[pl.ds(sid * 624, 624)])

            @pl.when(sid == 0)
            def _():
                pltpu.sync_copy(acc.at[pl.ds(9984, 16)],
                                out_h.at[chunk].at[pl.ds(9984, 16)])

            plsc.subcore_barrier()

    return k(vals, srcp, dstp)


def _scale_split_body(x_ref, dis_ref, o_ref):
    xd = x_ref[...] * dis_ref[...]
    o_ref[0] = xd[:, 0:128]
    o_ref[1] = xd[:, 128:256]


def _scale_split(x, dis):
    return pl.pallas_call(
        _scale_split_body,
        grid=(_N // _ROWS,),
        in_specs=[
            pl.BlockSpec((_ROWS, 256), lambda i: (i, 0)),
            pl.BlockSpec((_ROWS, 1), lambda i: (i, 0)),
        ],
        out_specs=pl.BlockSpec((2, _ROWS, 128), lambda i: (0, i, 0)),
        out_shape=jax.ShapeDtypeStruct((2, _N, 128), jnp.float32),
    )(x, dis)


def _mm_stats_body(agg_ref, xd_ref, dis_ref, w_ref, b_ref, y_ref, acc_ref):
    a = jnp.concatenate(
        [agg_ref[0] + xd_ref[0], agg_ref[1] + xd_ref[1]], axis=1)
    a = a * dis_ref[...]
    y = jnp.dot(a, w_ref[...], preferred_element_type=jnp.float32)
    y = y + b_ref[...][None, :]
    y_ref[...] = y

    @pl.when(pl.program_id(0) == 0)
    def _():
        acc_ref[...] = jnp.zeros_like(acc_ref)

    acc_ref[0:1, :] += jnp.sum(y, axis=0, keepdims=True)
    acc_ref[1:2, :] += jnp.sum(y * y, axis=0, keepdims=True)


def _mm_stats(agg, xd, dis, w, b):
    d = w.shape[1]
    return pl.pallas_call(
        _mm_stats_body,
        grid=(_N // _ROWS,),
        in_specs=[
            pl.BlockSpec((2, _ROWS, 128), lambda i: (0, i, 0)),
            pl.BlockSpec((2, _ROWS, 128), lambda i: (0, i, 0)),
            pl.BlockSpec((_ROWS, 1), lambda i: (i, 0)),
            pl.BlockSpec((256, d), lambda i: (0, 0)),
            pl.BlockSpec((d,), lambda i: (0,)),
        ],
        out_specs=[
            pl.BlockSpec((_ROWS, d), lambda i: (i, 0)),
            pl.BlockSpec((8, d), lambda i: (0, 0)),
        ],
        out_shape=[
            jax.ShapeDtypeStruct((_N, d), jnp.float32),
            jax.ShapeDtypeStruct((8, d), jnp.float32),
        ],
    )(agg, xd, dis, w, b)


def _bn_tanh_mm_body(y_ref, acc_ref, g_ref, be_ref, dis_ref, w_ref, p_ref):
    m = acc_ref[0:1, :] / _N
    v = acc_ref[1:2, :] / _N - m * m
    h = jnp.tanh((y_ref[...] - m) * lax.rsqrt(v + _EPS) * g_ref[...][None, :]
                 + be_ref[...][None, :])
    hd = h * dis_ref[...]
    p = jnp.dot(hd, w_ref[...], preferred_element_type=jnp.float32)
    for c in range(4):
        p_ref[c] = p[:, c * 128:(c + 1) * 128]


def _bn_tanh_mm(y, acc, g, be, dis, w):
    d = y.shape[1]
    do = w.shape[1]
    return pl.pallas_call(
        _bn_tanh_mm_body,
        grid=(_N // _ROWS,),
        in_specs=[
            pl.BlockSpec((_ROWS, d), lambda i: (i, 0)),
            pl.BlockSpec((8, d), lambda i: (0, 0)),
            pl.BlockSpec((d,), lambda i: (0,)),
            pl.BlockSpec((d,), lambda i: (0,)),
            pl.BlockSpec((_ROWS, 1), lambda i: (i, 0)),
            pl.BlockSpec((d, do), lambda i: (0, 0)),
        ],
        out_specs=pl.BlockSpec((4, _ROWS, 128), lambda i: (0, i, 0)),
        out_shape=jax.ShapeDtypeStruct((4, _N, 128), jnp.float32),
    )(y, acc, g, be, dis, w)


def _combine_stats_body(agg_ref, p_ref, dis_ref, b_ref, y_ref, acc_ref):
    y = jnp.concatenate([agg_ref[c] + p_ref[c] for c in range(4)], axis=1)
    y = y * dis_ref[...] + b_ref[...][None, :]
    y_ref[...] = y

    @pl.when(pl.program_id(0) == 0)
    def _():
        acc_ref[...] = jnp.zeros_like(acc_ref)

    acc_ref[0:1, :] += jnp.sum(y, axis=0, keepdims=True)
    acc_ref[1:2, :] += jnp.sum(y * y, axis=0, keepdims=True)


def _combine_stats(agg, p, dis, b):
    d = b.shape[0]
    return pl.pallas_call(
        _combine_stats_body,
        grid=(_N // _ROWS,),
        in_specs=[
            pl.BlockSpec((4, _ROWS, 128), lambda i: (0, i, 0)),
            pl.BlockSpec((4, _ROWS, 128), lambda i: (0, i, 0)),
            pl.BlockSpec((_ROWS, 1), lambda i: (i, 0)),
            pl.BlockSpec((d,), lambda i: (0,)),
        ],
        out_specs=[
            pl.BlockSpec((_ROWS, d), lambda i: (i, 0)),
            pl.BlockSpec((8, d), lambda i: (0, 0)),
        ],
        out_shape=[
            jax.ShapeDtypeStruct((_N, d), jnp.float32),
            jax.ShapeDtypeStruct((8, d), jnp.float32),
        ],
    )(agg, p, dis, b)


def _bn_tanh_pool_body(y_ref, acc_ref, g_ref, be_ref, batch_ref, s_ref,
                       cnt_ref):
    m = acc_ref[0:1, :] / _N
    v = acc_ref[1:2, :] / _N - m * m
    h = jnp.tanh((y_ref[...] - m) * lax.rsqrt(v + _EPS) * g_ref[...][None, :]
                 + be_ref[...][None, :])
    onehot = (batch_ref[...] ==
              jax.lax.broadcasted_iota(jnp.int32, (1, _G), 1)).astype(jnp.float32)

    @pl.when(pl.program_id(0) == 0)
    def _():
        s_ref[...] = jnp.zeros_like(s_ref)
        cnt_ref[...] = jnp.zeros_like(cnt_ref)

    dims = (((0,), (0,)), ((), ()))
    s_ref[...] += jax.lax.dot_general(onehot, h, dims,
                                      preferred_element_type=jnp.float32)
    cnt_ref[...] += jax.lax.dot_general(
        onehot, jnp.ones((onehot.shape[0], 128), jnp.float32), dims,
        preferred_element_type=jnp.float32)


def _bn_tanh_pool(y, acc, g, be, batch):
    d = y.shape[1]
    return pl.pallas_call(
        _bn_tanh_pool_body,
        grid=(_N // _ROWS,),
        in_specs=[
            pl.BlockSpec((_ROWS, d), lambda i: (i, 0)),
            pl.BlockSpec((8, d), lambda i: (0, 0)),
            pl.BlockSpec((d,), lambda i: (0,)),
            pl.BlockSpec((d,), lambda i: (0,)),
            pl.BlockSpec((_ROWS, 1), lambda i: (i, 0)),
        ],
        out_specs=[
            pl.BlockSpec((_G, d), lambda i: (0, 0)),
            pl.BlockSpec((_G, 128), lambda i: (0, 0)),
        ],
        out_shape=[
            jax.ShapeDtypeStruct((_G, d), jnp.float32),
            jax.ShapeDtypeStruct((_G, 128), jnp.float32),
        ],
    )(y, acc, g, be, batch)


def _head_body(s_ref, cnt_ref, wl_ref, bl_ref, gl_ref, bel_ref,
               wf_ref, bf_ref, gf_ref, bef_ref, o_ref):
    p = s_ref[...] / jnp.maximum(cnt_ref[:, 0:1], 1.0)
    y = jnp.dot(p, wl_ref[...], preferred_element_type=jnp.float32)
    y = y + bl_ref[...][None, :]
    m = jnp.mean(y, axis=0, keepdims=True)
    v = jnp.mean(y * y, axis=0, keepdims=True) - m * m
    z = jnp.tanh((y - m) * lax.rsqrt(v + _EPS) * gl_ref[...][None, :]
                 + bel_ref[...][None, :])
    o = jnp.dot(z, wf_ref[...], preferred_element_type=jnp.float32)
    o = o + bf_ref[...][None, :]
    m2 = jnp.mean(o, axis=0, keepdims=True)
    v2 = jnp.mean(o * o, axis=0, keepdims=True) - m2 * m2
    o = (o - m2) * lax.rsqrt(v2 + _EPS) * gf_ref[...][None, :] + bef_ref[...][None, :]
    o = o - jnp.max(o, axis=1, keepdims=True)
    o_ref[...] = o - jnp.log(jnp.sum(jnp.exp(o), axis=1, keepdims=True))


def _head(s, cnt, Wl, bl, gl, bel, Wf, bf, gf, bef):
    args = (s, cnt, Wl, bl, gl, bel, Wf, bf, gf, bef)
    return pl.pallas_call(
        _head_body,
        in_specs=[pl.BlockSpec(a.shape, (lambda *_, nd=a.ndim: (0,) * nd))
                  for a in args],
        out_specs=pl.BlockSpec((_G, Wf.shape[1]), lambda: (0, 0)),
        out_shape=jax.ShapeDtypeStruct((_G, Wf.shape[1]), jnp.float32),
    )(*args)


def kernel(x, edge_index, batch, W1, b1, g1, be1, W2, b2, g2, be2,
           Wl, bl, gl, bel, Wf, bf, gf, bef):
    src = edge_index[0]
    dst = edge_index[1]
    batch2 = batch[:, None]

    pad = _EPAD - _E
    srcp = jnp.concatenate([src, jnp.zeros((pad,), jnp.int32)]
                           ).reshape(_NSUB, _NBLK, _BLK)
    dstp = jnp.concatenate([dst, jnp.full((pad,), _N, jnp.int32)]
                           ).reshape(_NSUB, _NBLK, _BLK)

    degp = _deg_sc(dstp)
    deg = 1.0 + degp[0, :_N] + degp[1, :_N]
    dis2 = lax.rsqrt(deg)[:, None]

    xd = _scale_split(x, dis2)
    aggx = _seg_sum(xd, srcp, dstp)
    y1, acc1 = _mm_stats(aggx, xd, dis2, W1, b1)
    p2 = _bn_tanh_mm(y1, acc1, g1, be1, dis2, W2)
    agg2 = _seg_sum(p2, srcp, dstp)
    y2, acc2 = _combine_stats(agg2, p2, dis2, b2)
    s, cnt = _bn_tanh_pool(y2, acc2, g2, be2, batch2)
    return _head(s, cnt, Wl, bl, gl, bel, Wf, bf, gf, bef)
